# baseline (speedup 1.0000x reference)
import functools

import jax
import jax.numpy as jnp
from jax import lax
from jax.experimental import pallas as pl
from jax.experimental.pallas import tpu as pltpu
from jax.experimental.pallas import tpu_sc as plsc

_B, _S, _D = 4, 2048, 1024
_NW = 32
_LANES = 16

_S_SC = 768
_CH = _S_SC // _NW
_CHW = _CH * _D
_TC_BLK = 256


def _tc_add_body(x_ref, pos_ref, out_ref):
    out_ref[...] = x_ref[...] + pos_ref[...][None, :, :]


def _tc_add_tail(x, pos_table):
    blk0 = _S_SC // _TC_BLK
    return pl.pallas_call(
        _tc_add_body,
        grid=((_S - _S_SC) // _TC_BLK,),
        in_specs=[
            pl.BlockSpec((_B, _TC_BLK, _D), lambda i: (0, i + blk0, 0)),
            pl.BlockSpec((_TC_BLK, _D), lambda i: (i + blk0, 0)),
        ],
        out_specs=pl.BlockSpec((_B, _TC_BLK, _D), lambda i: (0, i + blk0, 0)),
        out_shape=jax.ShapeDtypeStruct((_B, _S, _D), x.dtype),
    )(x, pos_table)


def _sc_body(x_hbm, pos_hbm, out_hbm,
             pbuf, xbuf0, xbuf1, xbuf2,
             psem, lsem0, lsem1, lsem2, ssem0, ssem1, ssem2):
    wid = lax.axis_index("s") * 2 + lax.axis_index("c")
    xbufs = [xbuf0, xbuf1, xbuf2]
    lsems = [lsem0, lsem1, lsem2]
    ssems = [ssem0, ssem1, ssem2]
    row0 = wid * _CH

    jobs = list(range(_B))
    n = len(jobs)

    pos_h = pltpu.async_copy(pos_hbm.at[pl.ds(row0, _CH), :], pbuf, psem)
    load_h = [None] * n
    store_h = [None] * n
    for i in range(min(2, n)):
        load_h[i] = pltpu.async_copy(
            x_hbm.at[jobs[i], pl.ds(row0, _CH), :], xbufs[i % 3], lsems[i % 3])

    pos_h.wait()
    for i in range(n):
        b = jobs[i]
        xb = xbufs[i % 3]
        load_h[i].wait()

        @plsc.parallel_loop(0, _CHW, step=_LANES, unroll=8)
        def _(k):
            r = k // _D
            sl = pl.ds(k % _D, _LANES)
            xb[r, sl] = xb[r, sl] + pbuf[r, sl]

        store_h[i] = pltpu.async_copy(
            xb, out_hbm.at[b, pl.ds(row0, _CH), :], ssems[i % 3])
        if i + 2 < n:
            j = i + 2
            if j >= 3:
                store_h[j - 3].wait()
            load_h[j] = pltpu.async_copy(
                x_hbm.at[jobs[j], pl.ds(row0, _CH), :], xbufs[j % 3], lsems[j % 3])
    for i in range(max(0, n - 3), n):
        store_h[i].wait()


def _sc_add_head(x, pos_table):
    mesh = plsc.VectorSubcoreMesh(core_axis_name="c", subcore_axis_name="s")
    call = functools.partial(
        pl.kernel,
        mesh=mesh,
        out_type=jax.ShapeDtypeStruct((_B, _S_SC, _D), jnp.float32),
        scratch_types=[
            pltpu.VMEM((_CH, _D), jnp.float32),
            pltpu.VMEM((_CH, _D), jnp.float32),
            pltpu.VMEM((_CH, _D), jnp.float32),
            pltpu.VMEM((_CH, _D), jnp.float32),
            pltpu.SemaphoreType.DMA,
            pltpu.SemaphoreType.DMA,
            pltpu.SemaphoreType.DMA,
            pltpu.SemaphoreType.DMA,
            pltpu.SemaphoreType.DMA,
            pltpu.SemaphoreType.DMA,
            pltpu.SemaphoreType.DMA,
        ],
    )(_sc_body)
    return call(x, pos_table)


def kernel(x, pos_table):
    sc_out = _sc_add_head(x, pos_table)
    tc_out = _tc_add_tail(x, pos_table)
    return lax.dynamic_update_slice(tc_out, sc_out, (0, 0, 0))


# speedup vs baseline: 1.0918x; 1.0918x over previous
import functools

import jax
import jax.numpy as jnp
from jax import lax
from jax.experimental import pallas as pl
from jax.experimental.pallas import tpu as pltpu
from jax.experimental.pallas import tpu_sc as plsc

_B, _S, _D = 4, 2048, 1024
_NW = 32
_ROWS_PER_W = _S // _NW
_CH = 16
_CHW = _CH * _D
_LANES = 16
_RING = 4


def _tc_add_body(x_ref, pos_ref, out_ref):
    out_ref[...] = x_ref[...] + pos_ref[...][None, :, :]


def _tc_add(x, pos_table):
    B, S, D = x.shape
    S_BLK = 512
    return pl.pallas_call(
        _tc_add_body,
        grid=(S // S_BLK,),
        in_specs=[
            pl.BlockSpec((B, S_BLK, D), lambda i: (0, i, 0)),
            pl.BlockSpec((S_BLK, D), lambda i: (i, 0)),
        ],
        out_specs=pl.BlockSpec((B, S_BLK, D), lambda i: (0, i, 0)),
        out_shape=jax.ShapeDtypeStruct((B, S, D), x.dtype),
    )(x, pos_table)


def _sc_body(x_hbm, pos_hbm, out_hbm,
             pbuf0, pbuf1, xbuf0, xbuf1, xbuf2, xbuf3,
             psem0, psem1, lsem0, lsem1, lsem2, lsem3,
             ssem0, ssem1, ssem2, ssem3):
    wid = lax.axis_index("s") * 2 + lax.axis_index("c")
    pbufs, psems = [pbuf0, pbuf1], [psem0, psem1]
    xbufs = [xbuf0, xbuf1, xbuf2, xbuf3]
    lsems = [lsem0, lsem1, lsem2, lsem3]
    ssems = [ssem0, ssem1, ssem2, ssem3]
    n_ch = _ROWS_PER_W // _CH

    def seq0(c):
        return wid * _ROWS_PER_W + c * _CH

    jobs = [(c, b) for c in range(n_ch) for b in range(_B)]
    n = len(jobs)

    pos_h = [None] * n_ch
    pos_h[0] = pltpu.async_copy(
        pos_hbm.at[pl.ds(seq0(0), _CH), :], pbufs[0], psems[0])
    load_h = [None] * n
    store_h = [None] * n
    for i in range(min(_RING - 1, n)):
        c, b = jobs[i]
        load_h[i] = pltpu.async_copy(
            x_hbm.at[b, pl.ds(seq0(c), _CH), :],
            xbufs[i % _RING], lsems[i % _RING])

    for i in range(n):
        c, b = jobs[i]
        if b == 0:
            if c + 1 < n_ch:
                nc = c + 1
                pos_h[nc] = pltpu.async_copy(
                    pos_hbm.at[pl.ds(seq0(nc), _CH), :],
                    pbufs[nc % 2], psems[nc % 2])
            pos_h[c].wait()
        xb = xbufs[i % _RING]
        pb = pbufs[c % 2]
        load_h[i].wait()

        @plsc.parallel_loop(0, _CHW, step=_LANES, unroll=8)
        def _(k):
            r = k // _D
            sl = pl.ds(k % _D, _LANES)
            xb[r, sl] = xb[r, sl] + pb[r, sl]

        store_h[i] = pltpu.async_copy(
            xb, out_hbm.at[b, pl.ds(seq0(c), _CH), :], ssems[i % _RING])
        if i + _RING - 1 < n:
            j = i + _RING - 1
            if j >= _RING:
                store_h[j - _RING].wait()
            cj, bj = jobs[j]
            load_h[j] = pltpu.async_copy(
                x_hbm.at[bj, pl.ds(seq0(cj), _CH), :],
                xbufs[j % _RING], lsems[j % _RING])
    for i in range(max(0, n - _RING), n):
        store_h[i].wait()


def _sc_add(x, pos_table):
    mesh = plsc.VectorSubcoreMesh(core_axis_name="c", subcore_axis_name="s")
    call = functools.partial(
        pl.kernel,
        mesh=mesh,
        out_type=jax.ShapeDtypeStruct((_B, _S, _D), jnp.float32),
        scratch_types=(
            [pltpu.VMEM((_CH, _D), jnp.float32)] * 2
            + [pltpu.VMEM((_CH, _D), jnp.float32)] * _RING
            + [pltpu.SemaphoreType.DMA] * (2 + 2 * _RING)
        ),
    )(_sc_body)
    return call(x, pos_table)


def kernel(x, pos_table):
    return _sc_add(x, pos_table)


# speedup vs baseline: 2.1182x; 1.9401x over previous
import jax
import jax.numpy as jnp
from jax.experimental import pallas as pl


def _add_body(x_ref, pos_ref, out_ref):
    out_ref[...] = x_ref[...] + pos_ref[...][None, :, :]


def kernel(x, pos_table):
    B, S, D = x.shape
    S_BLK = 512
    return pl.pallas_call(
        _add_body,
        grid=(S // S_BLK,),
        in_specs=[
            pl.BlockSpec((B, S_BLK, D), lambda i: (0, i, 0)),
            pl.BlockSpec((S_BLK, D), lambda i: (i, 0)),
        ],
        out_specs=pl.BlockSpec((B, S_BLK, D), lambda i: (0, i, 0)),
        out_shape=jax.ShapeDtypeStruct((B, S, D), x.dtype),
    )(x, pos_table)
